# baseline (speedup 1.0000x reference)
import functools

import jax
import jax.numpy as jnp
from jax import lax
from jax.experimental import pallas as pl
from jax.experimental.pallas import tpu as pltpu
from jax.experimental.pallas import tpu_sc as plsc

_D = 64
_NW = 32
_NBUF = 2


def _body(nb, seq, idx_hbm, expr_hbm, table_hbm, out_hbm, idx_v, expr_v,
          g0, g1, s0, s1, gsem0, gsem1, ssem0, ssem1):
  nc = plsc.get_sparse_core_info().num_cores
  wid = lax.axis_index("s") * nc + lax.axis_index("c")
  base = wid * nb

  gbuf = (g0, g1)
  sbuf = (s0, s1)
  gsem = (gsem0, gsem1)
  ssem = (ssem0, ssem1)

  pltpu.sync_copy(idx_hbm.at[pl.ds(base, nb)], idx_v)
  pltpu.sync_copy(expr_hbm.at[pl.ds(base, nb)], expr_v)

  def gathers(i, b):
    return (
        pltpu.make_async_copy(table_hbm.at[idx_v.at[i, pl.ds(0, 128)]],
                              gbuf[b].at[pl.ds(0, 128)], gsem[b]),
        pltpu.make_async_copy(table_hbm.at[idx_v.at[i, pl.ds(128, seq - 128)]],
                              gbuf[b].at[pl.ds(128, seq - 128)], gsem[b]),
    )

  def scatter(i, b):
    return pltpu.make_async_copy(sbuf[b], out_hbm.at[base + i], ssem[b])

  def start_gathers(i, b):
    for c in gathers(i, b):
      c.start()

  def wait_gathers(i, b):
    for c in gathers(i, b):
      c.wait()

  for b in range(_NBUF):
    start_gathers(b, b)

  n_full = seq // 16
  tail = seq - n_full * 16
  tail_base = seq - 16

  def pack_rows(i, b):
    def group_body(g, _):
      ev = expr_v[i, pl.ds(g * 16, 16)]
      for r in range(16):
        e = ev[r]
        rr = g * 16 + r
        dr = g * 8 + r // 2
        for k in range(_D // 16):
          dl = pl.ds((r % 2) * _D + k * 16, 16)
          sbuf[b][dr, dl] = gbuf[b][rr, pl.ds(k * 16, 16)] * e
      return 0

    lax.fori_loop(0, n_full, group_body, 0)

    ev = expr_v[i, pl.ds(tail_base, 16)]
    for r in range(16 - tail, 16):
      e = ev[r]
      rr = tail_base + r
      dr = rr // 2
      for k in range(_D // 16):
        dl = pl.ds((r % 2) * _D + k * 16, 16)
        sbuf[b][dr, dl] = gbuf[b][rr, pl.ds(k * 16, 16)] * e

  def outer(io):
    for b in range(_NBUF):
      i = io + b
      wait_gathers(i, b)

      @pl.when(i >= _NBUF)
      def _():
        scatter(i - _NBUF, b).wait()

      pack_rows(i, b)
      scatter(i, b).start()

      @pl.when(i + _NBUF < nb)
      def _():
        start_gathers(i + _NBUF, b)

  pl.loop(0, nb, step=_NBUF)(outer)

  for b in range(_NBUF):
    scatter(nb - _NBUF + b, b).wait()


def _half(gene_indices, expression_values, embedding_table):
  bsz, seq = gene_indices.shape
  nb = bsz // _NW

  mesh = plsc.VectorSubcoreMesh(core_axis_name="c", subcore_axis_name="s")
  inter = pl.kernel(
      functools.partial(_body, nb, seq),
      out_type=jax.ShapeDtypeStruct((bsz, seq // 2, 2 * _D), jnp.float32),
      mesh=mesh,
      compiler_params=pltpu.CompilerParams(use_tc_tiling_on_sc=False),
      scratch_types=[
          pltpu.VMEM((nb, seq), jnp.int32),
          pltpu.VMEM((nb, seq), jnp.float32),
      ] + [pltpu.VMEM((seq, _D), jnp.float32)] * _NBUF
        + [pltpu.VMEM((seq // 2, 2 * _D), jnp.float32)] * _NBUF
        + [pltpu.SemaphoreType.DMA] * (2 * _NBUF),
  )(gene_indices, expression_values, embedding_table)

  return inter.reshape(bsz, seq, _D)


def kernel(gene_indices, expression_values, embedding_table):
  bsz, seq = gene_indices.shape
  assert bsz % (2 * _NW) == 0 and seq == 200
  h = bsz // 2
  idx = gene_indices.astype(jnp.int32)

  out_a = _half(idx[:h], expression_values[:h], embedding_table)
  out_b = _half(idx[h:], expression_values[h:], embedding_table)
  return jnp.concatenate([out_a, out_b], axis=0)


# speedup vs baseline: 1.1529x; 1.1529x over previous
import functools

import jax
import jax.numpy as jnp
from jax import lax
from jax.experimental import pallas as pl
from jax.experimental.pallas import tpu as pltpu
from jax.experimental.pallas import tpu_sc as plsc

_D = 64
_NW = 32
_NBUF = 2


def _body(nb, seq, idx_hbm, expr_hbm, table_hbm, out_hbm, idx_v, expr_v,
          g0, g1, s0, s1, gsem0, gsem1, ssem0, ssem1):
  nc = plsc.get_sparse_core_info().num_cores
  wid = lax.axis_index("s") * nc + lax.axis_index("c")
  base = wid * nb

  gbuf = (g0, g1)
  sbuf = (s0, s1)
  gsem = (gsem0, gsem1)
  ssem = (ssem0, ssem1)

  pltpu.sync_copy(idx_hbm.at[pl.ds(base * seq, nb * seq)], idx_v)
  pltpu.sync_copy(expr_hbm.at[pl.ds(base, nb)], expr_v)

  def gathers(i, b):
    return (
        pltpu.make_async_copy(table_hbm.at[idx_v.at[pl.ds(i * seq, 128)]],
                              gbuf[b].at[pl.ds(0, 128)], gsem[b]),
        pltpu.make_async_copy(
            table_hbm.at[idx_v.at[pl.ds(i * seq + 128, seq - 128)]],
            gbuf[b].at[pl.ds(128, seq - 128)], gsem[b]),
    )

  def scatter(i, b):
    return pltpu.make_async_copy(sbuf[b], out_hbm.at[base + i], ssem[b])

  def start_gathers(i, b):
    for c in gathers(i, b):
      c.start()

  def wait_gathers(i, b):
    for c in gathers(i, b):
      c.wait()

  for b in range(_NBUF):
    start_gathers(b, b)

  n_full = seq // 16
  tail = seq - n_full * 16
  tail_base = seq - 16

  def pack_rows(i, b):
    def group_body(g, _):
      ev = expr_v[i, pl.ds(g * 16, 16)]
      for r in range(16):
        e = ev[r]
        rr = g * 16 + r
        dr = g * 8 + r // 2
        for k in range(_D // 16):
          dl = pl.ds((r % 2) * _D + k * 16, 16)
          sbuf[b][dr, dl] = gbuf[b][rr, pl.ds(k * 16, 16)] * e
      return 0

    lax.fori_loop(0, n_full, group_body, 0)

    ev = expr_v[i, pl.ds(tail_base, 16)]
    for r in range(16 - tail, 16):
      e = ev[r]
      rr = tail_base + r
      dr = rr // 2
      for k in range(_D // 16):
        dl = pl.ds((r % 2) * _D + k * 16, 16)
        sbuf[b][dr, dl] = gbuf[b][rr, pl.ds(k * 16, 16)] * e

  def outer(io):
    for b in range(_NBUF):
      i = io + b
      wait_gathers(i, b)

      @pl.when(i >= _NBUF)
      def _():
        scatter(i - _NBUF, b).wait()

      pack_rows(i, b)
      scatter(i, b).start()

      @pl.when(i + _NBUF < nb)
      def _():
        start_gathers(i + _NBUF, b)

  pl.loop(0, nb, step=_NBUF)(outer)

  for b in range(_NBUF):
    scatter(nb - _NBUF + b, b).wait()


def kernel(gene_indices, expression_values, embedding_table):
  bsz, seq = gene_indices.shape
  assert bsz % _NW == 0 and seq == 200
  nb = bsz // _NW

  mesh = plsc.VectorSubcoreMesh(core_axis_name="c", subcore_axis_name="s")
  inter = pl.kernel(
      functools.partial(_body, nb, seq),
      out_type=jax.ShapeDtypeStruct((bsz, seq // 2, 2 * _D), jnp.float32),
      mesh=mesh,
      compiler_params=pltpu.CompilerParams(use_tc_tiling_on_sc=False),
      scratch_types=[
          pltpu.VMEM((nb * seq,), jnp.int32),
          pltpu.VMEM((nb, seq), jnp.float32),
      ] + [pltpu.VMEM((seq, _D), jnp.float32)] * _NBUF
        + [pltpu.VMEM((seq // 2, 2 * _D), jnp.float32)] * _NBUF
        + [pltpu.SemaphoreType.DMA] * (2 * _NBUF),
  )(gene_indices.astype(jnp.int32).reshape(-1), expression_values,
    embedding_table)

  return inter.reshape(bsz, seq, _D)


# speedup vs baseline: 1.1545x; 1.0014x over previous
import functools

import jax
import jax.numpy as jnp
from jax import lax
from jax.experimental import pallas as pl
from jax.experimental.pallas import tpu as pltpu
from jax.experimental.pallas import tpu_sc as plsc

_D = 64
_NW = 32
_NBUF = 2


def _body(nb, seq, idx_hbm, expr_hbm, table_hbm, out_hbm, idx_v, expr_v,
          g0, g1, s0, s1, gsem0, gsem1, ssem0, ssem1):
  nc = plsc.get_sparse_core_info().num_cores
  wid = lax.axis_index("s") * nc + lax.axis_index("c")
  base = wid * nb

  gbuf = (g0, g1)
  sbuf = (s0, s1)
  gsem = (gsem0, gsem1)
  ssem = (ssem0, ssem1)

  pltpu.sync_copy(idx_hbm.at[pl.ds(base * seq, nb * seq)], idx_v)
  pltpu.sync_copy(expr_hbm.at[pl.ds(base, nb)], expr_v)

  _SP = 104
  def gathers(i, b):
    return (
        pltpu.make_async_copy(table_hbm.at[idx_v.at[pl.ds(i * seq, _SP)]],
                              gbuf[b].at[pl.ds(0, _SP)], gsem[b]),
        pltpu.make_async_copy(
            table_hbm.at[idx_v.at[pl.ds(i * seq + _SP, seq - _SP)]],
            gbuf[b].at[pl.ds(_SP, seq - _SP)], gsem[b]),
    )

  def scatter(i, b):
    return pltpu.make_async_copy(sbuf[b], out_hbm.at[base + i], ssem[b])

  def start_gathers(i, b):
    for c in gathers(i, b):
      c.start()

  def wait_gathers(i, b):
    for c in gathers(i, b):
      c.wait()

  for b in range(_NBUF):
    start_gathers(b, b)

  n_full = seq // 16
  tail = seq - n_full * 16
  tail_base = seq - 16

  def pack_rows(i, b):
    def group_body(g, _):
      ev = expr_v[i, pl.ds(g * 16, 16)]
      for r in range(16):
        e = ev[r]
        rr = g * 16 + r
        dr = g * 8 + r // 2
        for k in range(_D // 16):
          dl = pl.ds((r % 2) * _D + k * 16, 16)
          sbuf[b][dr, dl] = gbuf[b][rr, pl.ds(k * 16, 16)] * e
      return 0

    lax.fori_loop(0, n_full, group_body, 0)

    ev = expr_v[i, pl.ds(tail_base, 16)]
    for r in range(16 - tail, 16):
      e = ev[r]
      rr = tail_base + r
      dr = rr // 2
      for k in range(_D // 16):
        dl = pl.ds((r % 2) * _D + k * 16, 16)
        sbuf[b][dr, dl] = gbuf[b][rr, pl.ds(k * 16, 16)] * e

  def outer(io):
    for b in range(_NBUF):
      i = io + b
      wait_gathers(i, b)

      @pl.when(i >= _NBUF)
      def _():
        scatter(i - _NBUF, b).wait()

      pack_rows(i, b)
      scatter(i, b).start()

      @pl.when(i + _NBUF < nb)
      def _():
        start_gathers(i + _NBUF, b)

  pl.loop(0, nb, step=_NBUF)(outer)

  for b in range(_NBUF):
    scatter(nb - _NBUF + b, b).wait()


def kernel(gene_indices, expression_values, embedding_table):
  bsz, seq = gene_indices.shape
  assert bsz % _NW == 0 and seq == 200
  nb = bsz // _NW

  mesh = plsc.VectorSubcoreMesh(core_axis_name="c", subcore_axis_name="s")
  inter = pl.kernel(
      functools.partial(_body, nb, seq),
      out_type=jax.ShapeDtypeStruct((bsz, seq // 2, 2 * _D), jnp.float32),
      mesh=mesh,
      compiler_params=pltpu.CompilerParams(use_tc_tiling_on_sc=False),
      scratch_types=[
          pltpu.VMEM((nb * seq,), jnp.int32),
          pltpu.VMEM((nb, seq), jnp.float32),
      ] + [pltpu.VMEM((seq, _D), jnp.float32)] * _NBUF
        + [pltpu.VMEM((seq // 2, 2 * _D), jnp.float32)] * _NBUF
        + [pltpu.SemaphoreType.DMA] * (2 * _NBUF),
  )(gene_indices.astype(jnp.int32).reshape(-1), expression_values,
    embedding_table)

  return inter.reshape(bsz, seq, _D)
